# baseline (speedup 1.0000x reference)
import functools

import jax
import jax.numpy as jnp
from jax import lax
from jax.experimental import pallas as pl
from jax.experimental.pallas import tpu as pltpu
from jax.experimental.pallas import tpu_sc as plsc

NUM_CORES = 2
NUM_SUBCORES = 16
NUM_WORKERS = NUM_CORES * NUM_SUBCORES

BATCH = 1024
CTX_DIM = 512
N_PRE = 5
N_CLS = 4
N_SUF = 68
N_TOK = N_PRE + N_CLS + N_SUF

ROWS_PER_WORKER = BATCH // NUM_WORKERS
GATHER_CHUNK = 8

ROW_BLOCK = 32


def _sc_gather(lab, table):
    mesh = plsc.VectorSubcoreMesh(
        core_axis_name="c",
        subcore_axis_name="s",
        num_cores=NUM_CORES,
        num_subcores=NUM_SUBCORES,
    )

    @functools.partial(
        pl.kernel,
        out_type=jax.ShapeDtypeStruct((BATCH, N_CLS, CTX_DIM), jnp.float32),
        mesh=mesh,
        scratch_types=[
            pltpu.VMEM((ROWS_PER_WORKER,), jnp.int32),
            pltpu.VMEM((ROWS_PER_WORKER, N_CLS, CTX_DIM), jnp.float32),
            pltpu.SemaphoreType.DMA,
            pltpu.SemaphoreType.DMA,
        ],
    )
    def sc_fill(lab_hbm, table_hbm, cls_hbm, idx_v, buf_a, gsem, csem):
        wid = lax.axis_index("s") * NUM_CORES + lax.axis_index("c")
        base = wid * ROWS_PER_WORKER
        pltpu.sync_copy(lab_hbm.at[pl.ds(base, ROWS_PER_WORKER)], idx_v)
        pltpu.async_copy(table_hbm.at[idx_v], buf_a, gsem).wait()
        pltpu.async_copy(
            buf_a, cls_hbm.at[pl.ds(base, ROWS_PER_WORKER)], csem).wait()

    return sc_fill(lab, table)


def _tc_assemble(cls_all, token_prefix, token_suffix):
    n_steps = BATCH // ROW_BLOCK

    def body(pre_ref, suf_ref, cls_ref, out_hbm, buf0, buf1, sem0, sem1):
        i = pl.program_id(0)

        def init_template(b):
            b[pl.ds(0, N_PRE)] = jnp.broadcast_to(
                pre_ref[0][:, None, :], (N_PRE, ROW_BLOCK, CTX_DIM))
            b[pl.ds(N_PRE + N_CLS, N_SUF)] = jnp.broadcast_to(
                suf_ref[0][:, None, :], (N_SUF, ROW_BLOCK, CTX_DIM))

        @pl.when(i == 0)
        def _():
            init_template(buf0)

        @pl.when(i == 1)
        def _():
            init_template(buf1)

        def step(buf, sem):
            @pl.when(i >= 2)
            def _():
                pltpu.make_async_copy(
                    buf, out_hbm.at[:, pl.ds((i - 2) * ROW_BLOCK, ROW_BLOCK), :],
                    sem).wait()
            buf[pl.ds(N_PRE, N_CLS)] = jnp.transpose(cls_ref[...], (1, 0, 2))
            pltpu.make_async_copy(
                buf, out_hbm.at[:, pl.ds(i * ROW_BLOCK, ROW_BLOCK), :],
                sem).start()

        @pl.when(i % 2 == 0)
        def _():
            step(buf0, sem0)

        @pl.when(i % 2 == 1)
        def _():
            step(buf1, sem1)

        @pl.when(i == n_steps - 1)
        def _():
            pltpu.make_async_copy(
                buf0, out_hbm.at[:, pl.ds((n_steps - 2) * ROW_BLOCK,
                                          ROW_BLOCK), :], sem0).wait()
            pltpu.make_async_copy(
                buf1, out_hbm.at[:, pl.ds((n_steps - 1) * ROW_BLOCK,
                                          ROW_BLOCK), :], sem1).wait()

    out_tm = pl.pallas_call(
        body,
        grid=(n_steps,),
        in_specs=[
            pl.BlockSpec((1, N_PRE, CTX_DIM), lambda i: (0, 0, 0)),
            pl.BlockSpec((1, N_SUF, CTX_DIM), lambda i: (0, 0, 0)),
            pl.BlockSpec((ROW_BLOCK, N_CLS, CTX_DIM), lambda i: (i, 0, 0)),
        ],
        out_specs=pl.BlockSpec(memory_space=pltpu.MemorySpace.HBM),
        out_shape=jax.ShapeDtypeStruct((N_TOK, BATCH, CTX_DIM), jnp.float32),
        scratch_shapes=[
            pltpu.VMEM((N_TOK, ROW_BLOCK, CTX_DIM), jnp.float32),
            pltpu.VMEM((N_TOK, ROW_BLOCK, CTX_DIM), jnp.float32),
            pltpu.SemaphoreType.DMA,
            pltpu.SemaphoreType.DMA,
        ],
    )(token_prefix, token_suffix, cls_all)
    return jnp.transpose(out_tm, (1, 0, 2))


def kernel(label, cls_ctx, token_prefix, token_suffix):
    lab = label.astype(jnp.int32)
    cls_all = _sc_gather(lab, cls_ctx)
    return _tc_assemble(cls_all, token_prefix, token_suffix)


# speedup vs baseline: 1.0080x; 1.0080x over previous
import functools

import jax
import jax.numpy as jnp
from jax import lax
from jax.experimental import pallas as pl
from jax.experimental.pallas import tpu as pltpu
from jax.experimental.pallas import tpu_sc as plsc

NUM_CORES = 2
NUM_SUBCORES = 16
NUM_WORKERS = NUM_CORES * NUM_SUBCORES

BATCH = 1024
CTX_DIM = 512
N_PRE = 5
N_CLS = 4
N_SUF = 68
N_TOK = N_PRE + N_CLS + N_SUF

ROWS_PER_WORKER = BATCH // NUM_WORKERS
GATHER_CHUNK = 8

ROW_BLOCK = 64


def _sc_gather(lab, table):
    mesh = plsc.VectorSubcoreMesh(
        core_axis_name="c",
        subcore_axis_name="s",
        num_cores=NUM_CORES,
        num_subcores=NUM_SUBCORES,
    )

    @functools.partial(
        pl.kernel,
        out_type=jax.ShapeDtypeStruct((BATCH, N_CLS, CTX_DIM), jnp.float32),
        mesh=mesh,
        scratch_types=[
            pltpu.VMEM((ROWS_PER_WORKER,), jnp.int32),
            pltpu.VMEM((ROWS_PER_WORKER, N_CLS, CTX_DIM), jnp.float32),
            pltpu.SemaphoreType.DMA,
            pltpu.SemaphoreType.DMA,
        ],
    )
    def sc_fill(lab_hbm, table_hbm, cls_hbm, idx_v, buf_a, gsem, csem):
        wid = lax.axis_index("s") * NUM_CORES + lax.axis_index("c")
        base = wid * ROWS_PER_WORKER
        pltpu.sync_copy(lab_hbm.at[pl.ds(base, ROWS_PER_WORKER)], idx_v)
        pltpu.async_copy(table_hbm.at[idx_v], buf_a, gsem).wait()
        pltpu.async_copy(
            buf_a, cls_hbm.at[pl.ds(base, ROWS_PER_WORKER)], csem).wait()

    return sc_fill(lab, table)


def _tc_assemble(cls_all, token_prefix, token_suffix):
    n_steps = BATCH // ROW_BLOCK

    def body(pre_ref, suf_ref, cls_ref, out_hbm, buf0, buf1, sem0, sem1):
        i = pl.program_id(0)

        def init_template(b):
            b[pl.ds(0, N_PRE)] = jnp.broadcast_to(
                pre_ref[0][:, None, :], (N_PRE, ROW_BLOCK, CTX_DIM))
            b[pl.ds(N_PRE + N_CLS, N_SUF)] = jnp.broadcast_to(
                suf_ref[0][:, None, :], (N_SUF, ROW_BLOCK, CTX_DIM))

        @pl.when(i == 0)
        def _():
            init_template(buf0)

        @pl.when(i == 1)
        def _():
            init_template(buf1)

        def step(buf, sem):
            @pl.when(i >= 2)
            def _():
                pltpu.make_async_copy(
                    buf, out_hbm.at[:, pl.ds((i - 2) * ROW_BLOCK, ROW_BLOCK), :],
                    sem).wait()
            buf[pl.ds(N_PRE, N_CLS)] = jnp.transpose(cls_ref[...], (1, 0, 2))
            pltpu.make_async_copy(
                buf, out_hbm.at[:, pl.ds(i * ROW_BLOCK, ROW_BLOCK), :],
                sem).start()

        @pl.when(i % 2 == 0)
        def _():
            step(buf0, sem0)

        @pl.when(i % 2 == 1)
        def _():
            step(buf1, sem1)

        @pl.when(i == n_steps - 1)
        def _():
            pltpu.make_async_copy(
                buf0, out_hbm.at[:, pl.ds((n_steps - 2) * ROW_BLOCK,
                                          ROW_BLOCK), :], sem0).wait()
            pltpu.make_async_copy(
                buf1, out_hbm.at[:, pl.ds((n_steps - 1) * ROW_BLOCK,
                                          ROW_BLOCK), :], sem1).wait()

    out_tm = pl.pallas_call(
        body,
        grid=(n_steps,),
        in_specs=[
            pl.BlockSpec((1, N_PRE, CTX_DIM), lambda i: (0, 0, 0)),
            pl.BlockSpec((1, N_SUF, CTX_DIM), lambda i: (0, 0, 0)),
            pl.BlockSpec((ROW_BLOCK, N_CLS, CTX_DIM), lambda i: (i, 0, 0)),
        ],
        out_specs=pl.BlockSpec(memory_space=pltpu.MemorySpace.HBM),
        out_shape=jax.ShapeDtypeStruct((N_TOK, BATCH, CTX_DIM), jnp.float32),
        scratch_shapes=[
            pltpu.VMEM((N_TOK, ROW_BLOCK, CTX_DIM), jnp.float32),
            pltpu.VMEM((N_TOK, ROW_BLOCK, CTX_DIM), jnp.float32),
            pltpu.SemaphoreType.DMA,
            pltpu.SemaphoreType.DMA,
        ],
    )(token_prefix, token_suffix, cls_all)
    return jnp.transpose(out_tm, (1, 0, 2))


def kernel(label, cls_ctx, token_prefix, token_suffix):
    lab = label.astype(jnp.int32)
    cls_all = _sc_gather(lab, cls_ctx)
    return _tc_assemble(cls_all, token_prefix, token_suffix)
